# baseline (speedup 1.0000x reference)
import functools

import jax
import jax.numpy as jnp
import numpy as np
from jax import lax
from jax.experimental import pallas as pl
from jax.experimental.pallas import tpu as pltpu
from jax.experimental.pallas import tpu_sc as plsc

N_SPECIAL = 25
S = 20
KR = 400
GW = 128


def _gelu_exact(x):
    return 0.5 * x * (1.0 + lax.erf(x * np.float32(1.0 / np.sqrt(2.0))))


def _table_body(fe_ref, lnw_ref, lnb_ref, w1_ref, b1_ref, w2_ref, b2_ref,
                out_ref):
    fe = fe_ref[...]
    mu = jnp.mean(fe, axis=1, keepdims=True)
    var = jnp.mean(fe * fe, axis=1, keepdims=True) - mu * mu
    xn = (fe - mu) * lax.rsqrt(var + 1e-5) * lnw_ref[...] + lnb_ref[...]
    h = lax.dot_general(xn.astype(jnp.bfloat16),
                        w1_ref[...].astype(jnp.bfloat16),
                        (((1,), (0,)), ((), ())),
                        preferred_element_type=jnp.float32) + b1_ref[...]
    h = _gelu_exact(h)
    logits = jnp.sum(h * w2_ref[...], axis=1, keepdims=True) + b2_ref[...]
    gate = jax.nn.sigmoid(logits)
    out_ref[...] = jnp.broadcast_to(gate, (KR, GW))


def _sc_gather_body(table_hbm, ids_hbm, out_hbm, ids_v, idx_v, rows_v, sem,
                    *, nc, chunk):
    wid = lax.axis_index("s") * nc + lax.axis_index("c")
    base = wid * chunk
    pltpu.sync_copy(ids_hbm.at[pl.ds(base, chunk)], ids_v)
    for r in range(chunk // 16):
        v = ids_v[pl.ds(r * 16, 16)]
        idx_v[pl.ds(r * 16, 16)] = jnp.clip(v - N_SPECIAL, 0, KR - 1)
    pltpu.async_copy(table_hbm.at[idx_v], rows_v, sem).wait()
    pltpu.sync_copy(rows_v, out_hbm.at[pl.ds(base, chunk)])


def _sc_gate_gather(table, ids_flat):
    n_tok = ids_flat.shape[0]
    info = plsc.get_sparse_core_info()
    nc, ns = info.num_cores, info.num_subcores
    chunk = n_tok // (nc * ns)
    mesh = plsc.VectorSubcoreMesh(core_axis_name="c", subcore_axis_name="s")
    return pl.kernel(
        functools.partial(_sc_gather_body, nc=nc, chunk=chunk),
        mesh=mesh,
        out_type=jax.ShapeDtypeStruct((n_tok, GW), jnp.float32),
        scratch_types=[
            pltpu.VMEM((chunk,), jnp.int32),
            pltpu.VMEM((chunk,), jnp.int32),
            pltpu.VMEM((chunk, GW), jnp.float32),
            pltpu.SemaphoreType.DMA,
        ],
    )(table, ids_flat)


def _main_body(ids_ref, gate_ref, shp_ref, sp_ref, fe_ref, out_ref,
               feb_scr, *, msl):
    i = pl.program_id(0)
    L = out_ref.shape[1]

    @pl.when(i == 0)
    def _cast():
        feb_scr[...] = fe_ref[...].astype(jnp.bfloat16)

    ids = ids_ref[0]
    idr = ids - N_SPECIAL
    gate = gate_ref[0][:, :1]
    krow = lax.broadcasted_iota(jnp.int32, (1, KR), 1)
    smask = idr == krow
    spec = ids < N_SPECIAL
    jcol = lax.broadcasted_iota(jnp.int32, (L, 1), 0)
    regv = jnp.logical_and(jnp.logical_not(spec),
                           jnp.logical_and(jcol >= 1, jcol <= msl))
    regf = regv.astype(jnp.float32)
    ca = regf * (1.0 - gate)
    cb = regf * gate
    shp_shift = jnp.concatenate(
        [jnp.zeros((1, S), jnp.float32), shp_ref[0],
         jnp.zeros((L - 1 - msl, S), jnp.float32)], axis=0)
    kcol = lax.broadcasted_iota(jnp.int32, (KR, 1), 0)
    srow = lax.broadcasted_iota(jnp.int32, (1, S), 1)
    rrep = (kcol % S == srow).astype(jnp.float32)
    shpt = lax.dot_general(shp_shift, rrep, (((1,), (1,)), ((), ())),
                           preferred_element_type=jnp.float32)
    mmask = jnp.maximum(idr, 0) // S == krow // S
    wreg = jnp.where(smask, ca, 0.0) + jnp.where(mmask, cb * shpt, 0.0)
    k26 = lax.broadcasted_iota(jnp.int32, (1, sp_ref.shape[0]), 1)
    wsp = jnp.logical_and(ids == k26, spec).astype(jnp.bfloat16)
    out_ref[0] = (
        lax.dot_general(wreg.astype(jnp.bfloat16), feb_scr[...],
                        (((1,), (0,)), ((), ())),
                        preferred_element_type=jnp.float32)
        + lax.dot_general(wsp, sp_ref[...].astype(jnp.bfloat16),
                          (((1,), (0,)), ((), ())),
                          preferred_element_type=jnp.float32))


def kernel(input_ids, shp_tensor, special_embedding, full_embed,
           ln_w, ln_b, W1, b1, W2, b2):
    B, L = input_ids.shape
    D = special_embedding.shape[1]
    n_tok = B * L
    shp_len = shp_tensor.shape[1]
    msl = min(shp_len, L - 2)
    nsp = special_embedding.shape[0]
    fe = full_embed.reshape(KR, D)

    gate_table = pl.pallas_call(
        _table_body,
        in_specs=[pl.BlockSpec((KR, D), lambda: (0, 0)),
                  pl.BlockSpec((1, D), lambda: (0, 0)),
                  pl.BlockSpec((1, D), lambda: (0, 0)),
                  pl.BlockSpec((D, D), lambda: (0, 0)),
                  pl.BlockSpec((1, D), lambda: (0, 0)),
                  pl.BlockSpec((1, D), lambda: (0, 0)),
                  pl.BlockSpec((1, 1), lambda: (0, 0))],
        out_specs=pl.BlockSpec((KR, GW), lambda: (0, 0)),
        out_shape=jax.ShapeDtypeStruct((KR, GW), jnp.float32),
    )(fe, ln_w.reshape(1, D), ln_b.reshape(1, D), W1, b1.reshape(1, D),
      W2.reshape(1, D), b2.reshape(1, 1))

    gate_rows = _sc_gate_gather(gate_table, input_ids.reshape(n_tok))
    gate_rows = gate_rows.reshape(B, L, GW)

    return pl.pallas_call(
        functools.partial(_main_body, msl=msl),
        grid=(B,),
        in_specs=[
            pl.BlockSpec((1, L, 1), lambda i: (i, 0, 0)),
            pl.BlockSpec((1, L, GW), lambda i: (i, 0, 0)),
            pl.BlockSpec((1, shp_len, S), lambda i: (i, 0, 0)),
            pl.BlockSpec((nsp, D), lambda i: (0, 0)),
            pl.BlockSpec((KR, D), lambda i: (0, 0)),
        ],
        out_specs=pl.BlockSpec((1, L, D), lambda i: (i, 0, 0)),
        out_shape=jax.ShapeDtypeStruct((B, L, D), jnp.float32),
        scratch_shapes=[pltpu.VMEM((KR, D), jnp.bfloat16)],
    )(input_ids[..., None], gate_rows, shp_tensor, special_embedding, fe)


# speedup vs baseline: 2.5718x; 2.5718x over previous
import functools

import jax
import jax.numpy as jnp
import numpy as np
from jax import lax
from jax.experimental import pallas as pl
from jax.experimental.pallas import tpu as pltpu

N_SPECIAL = 25
S = 20
KR = 400
T = 512


def _gelu_exact(x):
    return 0.5 * x * (1.0 + lax.erf(x * np.float32(1.0 / np.sqrt(2.0))))


def _body(ids_ref, shp_ref, sp_ref, fe_ref, lnw_ref, lnb_ref, w1_ref,
          b1_ref, w2_ref, b2_ref, out_ref, gate_scr, feb_scr, *, msl):
    b = pl.program_id(0)
    tb = pl.program_id(1)

    @pl.when(jnp.logical_and(b == 0, tb == 0))
    def _table():
        for q in range(fe_ref.shape[0]):
            feb_scr[pl.ds(q * S, S), :] = fe_ref[q].astype(jnp.bfloat16)
        fe = feb_scr[...].astype(jnp.float32)
        mu = jnp.mean(fe, axis=1, keepdims=True)
        var = jnp.mean(fe * fe, axis=1, keepdims=True) - mu * mu
        xn = (fe - mu) * lax.rsqrt(var + 1e-5) * lnw_ref[...] + lnb_ref[...]
        h = lax.dot_general(xn.astype(jnp.bfloat16),
                            w1_ref[...].astype(jnp.bfloat16),
                            (((1,), (0,)), ((), ())),
                            preferred_element_type=jnp.float32) + b1_ref[...]
        h = _gelu_exact(h)
        logits = lax.dot_general(w2_ref[...], h, (((1,), (1,)), ((), ())),
                                 preferred_element_type=jnp.float32)
        gate_scr[...] = jax.nn.sigmoid(logits + b2_ref[...])

    ids = ids_ref[...]
    idr = ids - N_SPECIAL
    kcol = lax.broadcasted_iota(jnp.int32, (KR, 1), 0)
    smask = kcol == idr
    sb = smask.astype(jnp.bfloat16)
    gate = lax.dot_general(gate_scr[...].astype(jnp.bfloat16), sb,
                           (((1,), (0,)), ((), ())),
                           preferred_element_type=jnp.float32)
    spec = ids < N_SPECIAL
    jrow = tb * T + lax.broadcasted_iota(jnp.int32, (1, T), 1)
    regv = jnp.logical_and(jnp.logical_not(spec),
                           jnp.logical_and(jrow >= 1, jrow <= msl))
    regf = regv.astype(jnp.float32)
    ca = (regf * (1.0 - gate)).astype(jnp.bfloat16)
    cb = (regf * gate).astype(jnp.bfloat16)
    srow = lax.broadcasted_iota(jnp.int32, (1, S), 1)
    rrep = (kcol % S == srow).astype(jnp.bfloat16)
    shpt = lax.dot_general(rrep, shp_ref[0].astype(jnp.bfloat16),
                           (((1,), (1,)), ((), ())),
                           preferred_element_type=jnp.float32
                           ).astype(jnp.bfloat16)
    mmask = kcol // S == jnp.maximum(idr, 0) // S
    wreg = sb * ca + mmask.astype(jnp.bfloat16) * (cb * shpt)
    k26 = lax.broadcasted_iota(jnp.int32, (sp_ref.shape[0], 1), 0)
    wsp = jnp.logical_and(k26 == ids, spec).astype(jnp.bfloat16)
    out_ref[0] = (
        lax.dot_general(wreg, feb_scr[...],
                        (((0,), (0,)), ((), ())),
                        preferred_element_type=jnp.float32)
        + lax.dot_general(wsp, sp_ref[...].astype(jnp.bfloat16),
                          (((0,), (0,)), ((), ())),
                          preferred_element_type=jnp.float32))


def kernel(input_ids, shp_tensor, special_embedding, full_embed,
           ln_w, ln_b, W1, b1, W2, b2):
    B, L = input_ids.shape
    D = special_embedding.shape[1]
    msl = min(shp_tensor.shape[1], L - 2)
    nsp = special_embedding.shape[0]
    nt = L // T
    shp_pad = jnp.pad(shp_tensor, ((0, 0), (1, L - 1 - msl), (0, 0)))
    ids2 = input_ids.reshape(1, B * L)

    return pl.pallas_call(
        functools.partial(_body, msl=msl),
        grid=(B, nt),
        in_specs=[
            pl.BlockSpec((1, T), lambda b, t: (0, b * nt + t)),
            pl.BlockSpec((1, T, S), lambda b, t: (b, t, 0)),
            pl.BlockSpec((nsp, D), lambda b, t: (0, 0)),
            pl.BlockSpec((S, S, D), lambda b, t: (0, 0, 0)),
            pl.BlockSpec((1, D), lambda b, t: (0, 0)),
            pl.BlockSpec((1, D), lambda b, t: (0, 0)),
            pl.BlockSpec((D, D), lambda b, t: (0, 0)),
            pl.BlockSpec((1, D), lambda b, t: (0, 0)),
            pl.BlockSpec((1, D), lambda b, t: (0, 0)),
            pl.BlockSpec((1, 1), lambda b, t: (0, 0)),
        ],
        out_specs=pl.BlockSpec((1, T, D), lambda b, t: (b, t, 0)),
        out_shape=jax.ShapeDtypeStruct((B, L, D), jnp.float32),
        scratch_shapes=[pltpu.VMEM((1, KR), jnp.float32),
                        pltpu.VMEM((KR, D), jnp.bfloat16)],
    )(ids2, shp_pad, special_embedding, full_embed,
      ln_w.reshape(1, D), ln_b.reshape(1, D), W1, b1.reshape(1, D),
      W2.reshape(1, D), b2.reshape(1, 1))


# speedup vs baseline: 2.5900x; 1.0070x over previous
import functools

import jax
import jax.numpy as jnp
import numpy as np
from jax import lax
from jax.experimental import pallas as pl
from jax.experimental.pallas import tpu as pltpu

N_SPECIAL = 25
S = 20
KR = 400
T = 512


def _gelu_exact(x):
    return 0.5 * x * (1.0 + lax.erf(x * np.float32(1.0 / np.sqrt(2.0))))


def _body(ids_ref, shp_ref, sp_ref, fe_ref, lnw_ref, lnb_ref, w1_ref,
          b1_ref, w2_ref, b2_ref, out_ref, gate_scr, feb_scr, *, msl):
    b = pl.program_id(0)
    tb = pl.program_id(1)

    @pl.when(jnp.logical_and(b == 0, tb == 0))
    def _table():
        for q in range(fe_ref.shape[0]):
            feb_scr[pl.ds(q * S, S), :] = fe_ref[q].astype(jnp.bfloat16)
        fe = feb_scr[...].astype(jnp.float32)
        mu = jnp.mean(fe, axis=1, keepdims=True)
        var = jnp.mean(fe * fe, axis=1, keepdims=True) - mu * mu
        xn = (fe - mu) * lax.rsqrt(var + 1e-5) * lnw_ref[...] + lnb_ref[...]
        h = lax.dot_general(xn.astype(jnp.bfloat16),
                            w1_ref[...].astype(jnp.bfloat16),
                            (((1,), (0,)), ((), ())),
                            preferred_element_type=jnp.float32) + b1_ref[...]
        h = _gelu_exact(h)
        logits = lax.dot_general(w2_ref[...], h, (((1,), (1,)), ((), ())),
                                 preferred_element_type=jnp.float32)
        gate_scr[...] = jax.nn.sigmoid(logits + b2_ref[...])

    ids = ids_ref[...]
    idr = ids - N_SPECIAL
    kcol = lax.broadcasted_iota(jnp.int32, (KR, 1), 0)
    smask = kcol == idr
    sb = smask.astype(jnp.bfloat16)
    gate = lax.dot_general(gate_scr[...].astype(jnp.bfloat16), sb,
                           (((1,), (0,)), ((), ())),
                           preferred_element_type=jnp.float32)
    spec = ids < N_SPECIAL
    jrow = tb * T + lax.broadcasted_iota(jnp.int32, (1, T), 1)
    regv = jnp.logical_and(jnp.logical_not(spec),
                           jnp.logical_and(jrow >= 1, jrow <= msl))
    regf = regv.astype(jnp.float32)
    ca = regf * (1.0 - gate)
    cb = regf * gate
    srow = lax.broadcasted_iota(jnp.int32, (1, S), 1)
    rrep = (kcol % S == srow).astype(jnp.float32)
    shpt = lax.dot_general(rrep, shp_ref[0], (((1,), (1,)), ((), ())),
                           preferred_element_type=jnp.float32)
    mmask = kcol // S == jnp.maximum(idr, 0) // S
    wreg = jnp.where(smask, ca, 0.0) + jnp.where(mmask, cb * shpt, 0.0)
    k26 = lax.broadcasted_iota(jnp.int32, (sp_ref.shape[0], 1), 0)
    wsp = jnp.logical_and(k26 == ids, spec).astype(jnp.bfloat16)
    out_ref[0] = (
        lax.dot_general(wreg.astype(jnp.bfloat16), feb_scr[...],
                        (((0,), (0,)), ((), ())),
                        preferred_element_type=jnp.float32)
        + lax.dot_general(wsp, sp_ref[...].astype(jnp.bfloat16),
                          (((0,), (0,)), ((), ())),
                          preferred_element_type=jnp.float32))


def kernel(input_ids, shp_tensor, special_embedding, full_embed,
           ln_w, ln_b, W1, b1, W2, b2):
    B, L = input_ids.shape
    D = special_embedding.shape[1]
    msl = min(shp_tensor.shape[1], L - 2)
    nsp = special_embedding.shape[0]
    nt = L // T
    shp_pad = jnp.pad(shp_tensor, ((0, 0), (1, L - 1 - msl), (0, 0)))
    ids2 = input_ids.reshape(1, B * L)

    return pl.pallas_call(
        functools.partial(_body, msl=msl),
        grid=(B, nt),
        in_specs=[
            pl.BlockSpec((1, T), lambda b, t: (0, b * nt + t)),
            pl.BlockSpec((1, T, S), lambda b, t: (b, t, 0)),
            pl.BlockSpec((nsp, D), lambda b, t: (0, 0)),
            pl.BlockSpec((S, S, D), lambda b, t: (0, 0, 0)),
            pl.BlockSpec((1, D), lambda b, t: (0, 0)),
            pl.BlockSpec((1, D), lambda b, t: (0, 0)),
            pl.BlockSpec((D, D), lambda b, t: (0, 0)),
            pl.BlockSpec((1, D), lambda b, t: (0, 0)),
            pl.BlockSpec((1, D), lambda b, t: (0, 0)),
            pl.BlockSpec((1, 1), lambda b, t: (0, 0)),
        ],
        out_specs=pl.BlockSpec((1, T, D), lambda b, t: (b, t, 0)),
        out_shape=jax.ShapeDtypeStruct((B, L, D), jnp.float32),
        scratch_shapes=[pltpu.VMEM((1, KR), jnp.float32),
                        pltpu.VMEM((KR, D), jnp.bfloat16)],
    )(ids2, shp_pad, special_embedding, full_embed,
      ln_w.reshape(1, D), ln_b.reshape(1, D), W1, b1.reshape(1, D),
      W2.reshape(1, D), b2.reshape(1, 1))
